# baseline (speedup 1.0000x reference)
import functools

import jax
import jax.numpy as jnp
from jax import lax
from jax.experimental import pallas as pl
from jax.experimental.pallas import tpu as pltpu
from jax.experimental.pallas import tpu_sc as plsc

_B, _N, _K = 2, 4096, 32
_D = 128


def _leaky(x):
    return jnp.where(x > 0, x, 0.1 * x)


def _prep_body(c1_ref, c2_ref, x2_ref, w0_ref, w1_ref, wc1_ref, wc2_ref,
               a1_ref, t1_ref):
    w0 = w0_ref[...]
    w1 = w1_ref[...]

    def feats(c, wc):
        h = (c[:, 0:1] * w0[0][None, :] + c[:, 1:2] * w0[1][None, :]
             + c[:, 2:3] * w0[2][None, :])
        f = _leaky(jnp.dot(_leaky(h), w1, preferred_element_type=jnp.float32))
        return jnp.dot(f, wc, preferred_element_type=jnp.float32)

    a1_ref[...] = feats(c1_ref[...], wc1_ref[...])
    a2 = feats(c2_ref[...], wc2_ref[...])
    pad = jnp.zeros((a2.shape[0], _D - 67), jnp.float32)
    t1_ref[...] = jnp.concatenate([x2_ref[...], a2, pad], axis=1)


def _prep(color1, color2, xyz2, W0, W1, Wc1, Wc2):
    R = _B * _N
    grid = 8
    blk = R // grid
    return pl.pallas_call(
        _prep_body,
        grid=(grid,),
        in_specs=[
            pl.BlockSpec((blk, 3), lambda i: (i, 0)),
            pl.BlockSpec((blk, 3), lambda i: (i, 0)),
            pl.BlockSpec((blk, 3), lambda i: (i, 0)),
            pl.BlockSpec((3, 32), lambda i: (0, 0)),
            pl.BlockSpec((32, 64), lambda i: (0, 0)),
            pl.BlockSpec((64, 64), lambda i: (0, 0)),
            pl.BlockSpec((64, 64), lambda i: (0, 0)),
        ],
        out_specs=[
            pl.BlockSpec((blk, 64), lambda i: (i, 0)),
            pl.BlockSpec((blk, _D), lambda i: (i, 0)),
        ],
        out_shape=[
            jax.ShapeDtypeStruct((R, 64), jnp.float32),
            jax.ShapeDtypeStruct((R, _D), jnp.float32),
        ],
    )(color1.reshape(R, 3), color2.reshape(R, 3), xyz2.reshape(R, 3),
      W0, W1, Wc1, Wc2)


_NS, _NL = 8, 512
_MAXKEY = 0x7FFFFFFF

_BATCHER8 = [(0, 1), (2, 3), (4, 5), (6, 7),
             (0, 2), (1, 3), (4, 6), (5, 7),
             (1, 2), (5, 6),
             (0, 4), (1, 5), (2, 6), (3, 7),
             (2, 4), (3, 5),
             (1, 2), (3, 4), (5, 6)]


def _knn_body(q_ref, db_ref, idx_ref):
    bb = pl.program_id(0)
    q = q_ref[0]
    R = q.shape[0]
    sqq = jnp.sum(q * q, axis=1, keepdims=True)
    keys = []
    for s in range(_NS):
        dbt = db_ref[0, :, s, :]
        sqdb = jnp.sum(dbt * dbt, axis=0, keepdims=True)
        dist = sqq + sqdb - 2.0 * jnp.dot(q, dbt, preferred_element_type=jnp.float32)
        bits = lax.bitcast_convert_type(jnp.maximum(dist, 0.0), jnp.int32)
        keys.append((bits & jnp.int32(-8)) | jnp.int32(s))
    for a, b in _BATCHER8:
        lo = jnp.minimum(keys[a], keys[b])
        hi = jnp.maximum(keys[a], keys[b])
        keys[a], keys[b] = lo, hi
    S = jnp.concatenate([keys[s][:, None, :] for s in range(_NS)], axis=1)
    heads = keys[0]
    pc = jnp.zeros((R, _NL), jnp.int32)
    iota512 = lax.broadcasted_iota(jnp.int32, (R, _NL), 1)
    iota = lax.broadcasted_iota(jnp.int32, (R, 128), 1)
    base = bb * _N
    out = []
    for _ in range(_K):
        h = [heads[:, t * 128:(t + 1) * 128] for t in range(4)]
        v01 = jnp.minimum(h[0], h[1])
        v23 = jnp.minimum(h[2], h[3])
        sh = jnp.minimum(v01, v23)
        ts = jnp.where(v01 <= v23,
                       jnp.where(h[0] <= h[1], 0, 1),
                       jnp.where(h[2] <= h[3], 2, 3))
        m = jnp.min(sh, axis=1, keepdims=True)
        g = jnp.min(jnp.where(sh == m, iota, jnp.int32(4096)),
                    axis=1, keepdims=True)
        gb = jnp.broadcast_to(g, ts.shape)
        tstar = jnp.take_along_axis(ts, gb, axis=1)[:, :1]
        sstar = m & jnp.int32(7)
        gg = tstar * jnp.int32(128) + g
        out.append(sstar * jnp.int32(_NL) + gg + base)
        pc = pc + (iota512 == gg).astype(jnp.int32)
        nxt = jnp.take_along_axis(S, jnp.minimum(pc, 7)[:, None, :],
                                  axis=1)[:, 0, :]
        heads = jnp.where(pc > 7, jnp.int32(_MAXKEY), nxt)
    idx_ref[0] = jnp.concatenate(out, axis=1)


def _knn(points_db, points_query):
    dbr = points_db.transpose(0, 2, 1).reshape(_B, 3, _NS, _NL)
    R = 512
    nb = _N // R
    return pl.pallas_call(
        _knn_body,
        grid=(_B, nb),
        in_specs=[
            pl.BlockSpec((1, R, 3), lambda b, i: (b, i, 0)),
            pl.BlockSpec((1, 3, _NS, _NL), lambda b, i: (b, 0, 0, 0)),
        ],
        out_specs=pl.BlockSpec((1, R, _K), lambda b, i: (b, i, 0)),
        out_shape=jax.ShapeDtypeStruct((_B, _N, _K), jnp.int32),
    )(points_query, dbr)


_NW = 32
_WIN = 256


def _sc_gather(table, idx):
    bt = idx.shape[0]
    bpw = bt // _NW
    nwin = bpw // _WIN
    mesh = plsc.VectorSubcoreMesh(core_axis_name="c", subcore_axis_name="s")

    @functools.partial(
        pl.kernel, mesh=mesh,
        out_type=jax.ShapeDtypeStruct((bt, _D), jnp.float32),
        scratch_types=[
            pltpu.VMEM((bpw,), jnp.int32),
            pltpu.VMEM((_WIN, _D), jnp.float32),
            pltpu.VMEM((_WIN, _D), jnp.float32),
            pltpu.SemaphoreType.DMA,
            pltpu.SemaphoreType.DMA,
        ],
    )
    def k(table_hbm, idx_hbm, out_hbm, idx_v, rows0, rows1, sem0, sem1):
        wid = lax.axis_index("s") * 2 + lax.axis_index("c")
        base = wid * bpw
        pltpu.sync_copy(idx_hbm.at[pl.ds(base, bpw)], idx_v)
        rows = (rows0, rows1)
        sems = (sem0, sem1)
        handles = [None, None]
        for w in range(nwin):
            b = w % 2
            handles[b] = pltpu.async_copy(
                table_hbm.at[idx_v.at[pl.ds(w * _WIN, _WIN)]], rows[b], sems[b])
            if w > 0:
                pb = (w - 1) % 2
                handles[pb].wait()
                pltpu.sync_copy(rows[pb],
                                out_hbm.at[pl.ds(base + (w - 1) * _WIN, _WIN)])
        lb = (nwin - 1) % 2
        handles[lb].wait()
        pltpu.sync_copy(rows[lb], out_hbm.at[pl.ds(base + (nwin - 1) * _WIN, _WIN)])

    return k(table, idx)


def _p1_body(x1_ref, a1_ref, g_ref, ww_ref, wc3_ref, t2_ref):
    x1 = x1_ref[...]
    g = g_ref[...]
    dirs = g[:, :, 0:3] - x1[:, None, :]
    dx = dirs[:, :, 0:1]
    dy = dirs[:, :, 1:2]
    dz = dirs[:, :, 2:3]
    ww = ww_ref[...]
    wl = (dx * ww[0][None, None, :] + dy * ww[1][None, None, :]
          + dz * ww[2][None, None, :])
    m = jnp.max(wl, axis=1, keepdims=True)
    e = jnp.exp(wl - m)
    den = jnp.sum(e, axis=1)
    wc3 = wc3_ref[...]
    dwc = (dx * wc3[0][None, None, :] + dy * wc3[1][None, None, :]
           + dz * wc3[2][None, None, :])
    cost = _leaky(a1_ref[...][:, None, :] + g[:, :, 3:67] + dwc)
    p2p = jnp.sum(e * cost, axis=1) / den
    pad = jnp.zeros((x1.shape[0], _D - 67), jnp.float32)
    t2_ref[...] = jnp.concatenate([x1, p2p, pad], axis=1)


def _phase1(x1, a1, g1, Ww, Wc3):
    R = _B * _N
    grid = 16
    blk = R // grid
    return pl.pallas_call(
        _p1_body,
        grid=(grid,),
        in_specs=[
            pl.BlockSpec((blk, 3), lambda i: (i, 0)),
            pl.BlockSpec((blk, 64), lambda i: (i, 0)),
            pl.BlockSpec((blk, _K, _D), lambda i: (i, 0, 0)),
            pl.BlockSpec((3, 64), lambda i: (0, 0)),
            pl.BlockSpec((3, 64), lambda i: (0, 0)),
        ],
        out_specs=pl.BlockSpec((blk, _D), lambda i: (i, 0)),
        out_shape=jax.ShapeDtypeStruct((R, _D), jnp.float32),
    )(x1, a1, g1, Ww, Wc3)


def _p2_body(x1_ref, g_ref, ww_ref, wfc_ref, out_ref):
    x1 = x1_ref[...]
    g = g_ref[...]
    dirs = g[:, :, 0:3] - x1[:, None, :]
    dx = dirs[:, :, 0:1]
    dy = dirs[:, :, 1:2]
    dz = dirs[:, :, 2:3]
    ww = ww_ref[...]
    wl = (dx * ww[0][None, None, :] + dy * ww[1][None, None, :]
          + dz * ww[2][None, None, :])
    m = jnp.max(wl, axis=1, keepdims=True)
    e = jnp.exp(wl - m)
    den = jnp.sum(e, axis=1)
    s = jnp.sum(e * g[:, :, 3:67], axis=1) / den
    flow = jnp.dot(s, wfc_ref[...], preferred_element_type=jnp.float32)
    out_ref[...] = jnp.clip(flow, -200.0, 200.0)


def _phase2(x1, g2, Ww, Wfc):
    R = _B * _N
    grid = 16
    blk = R // grid
    return pl.pallas_call(
        _p2_body,
        grid=(grid,),
        in_specs=[
            pl.BlockSpec((blk, 3), lambda i: (i, 0)),
            pl.BlockSpec((blk, _K, _D), lambda i: (i, 0, 0)),
            pl.BlockSpec((3, 64), lambda i: (0, 0)),
            pl.BlockSpec((64, 3), lambda i: (0, 0)),
        ],
        out_specs=pl.BlockSpec((blk, 3), lambda i: (i, 0)),
        out_shape=jax.ShapeDtypeStruct((R, 3), jnp.float32),
    )(x1, g2, Ww, Wfc)


def kernel(xyz1, xyz2, color1, color2, W0, W1, Wc, Ww, Wfc):
    R = _B * _N
    x1 = xyz1.reshape(R, 3)
    a1, t1 = _prep(color1, color2, xyz2, W0, W1, Wc[:64], Wc[64:128])

    idx1 = _knn(xyz2, xyz1).reshape(R * _K)
    idx2 = _knn(xyz1, xyz1).reshape(R * _K)
    g1 = _sc_gather(t1, idx1).reshape(R, _K, _D)
    t2 = _phase1(x1, a1, g1, Ww, Wc[128:131])

    g2 = _sc_gather(t2, idx2).reshape(R, _K, _D)
    flow = _phase2(x1, g2, Ww, Wfc)
    return flow.reshape(_B, _N, 3)
